# baseline (speedup 1.0000x reference)
import math

import jax
import jax.numpy as jnp
from jax.experimental import pallas as pl
from jax.experimental.pallas import tpu as pltpu

N = 2048
HEADS = 16
NUM_BUCKETS = 32
MAX_DISTANCE = 128
BLK_I = 128
WW = 4352
VW = 4096


def _body(tab_ref, o_ref, w_ref, v128_ref):
    t = pl.program_id(1)

    @pl.when(t == 0)
    def _compute_diag():
        s = jax.lax.broadcasted_iota(jnp.int32, (8, WW), 0)
        z = jax.lax.broadcasted_iota(jnp.int32, (8, WW), 1)
        d = jnp.clip(z - s - (121 + N - 1), -(N - 1), N - 1)
        nb = NUM_BUCKETS // 2
        neg = -d
        ret = jnp.where(neg < 0, nb, 0)
        an = jnp.abs(neg)
        max_exact = nb // 2
        nf = jnp.maximum(an.astype(jnp.float32), 1.0)
        val_large = max_exact + (
            jnp.log(nf / max_exact) / math.log(MAX_DISTANCE / max_exact) * (nb - max_exact)
        ).astype(jnp.int32)
        val_large = jnp.minimum(val_large, nb - 1)
        bucket = ret + jnp.where(an < max_exact, an, val_large)
        acc = jnp.zeros((8, WW), jnp.float32)
        for b in range(NUM_BUCKETS):
            acc = jnp.where(bucket == b, tab_ref[0, 0, b], acc)
        w_ref[:, :] = acc
        for k in range(16):
            v128_ref[8 * k : 8 * k + 8, :] = w_ref[:, 120 - 8 * k : 120 - 8 * k + VW]

    o_ref[0, :, :] = v128_ref[:, pl.ds((16 - t) * BLK_I, N)]


def kernel(n, relative_attention_bias):
    del n
    tab_t = relative_attention_bias.T.reshape(HEADS, 1, NUM_BUCKETS)
    out = pl.pallas_call(
        _body,
        grid=(HEADS, N // BLK_I),
        in_specs=[pl.BlockSpec((1, 1, NUM_BUCKETS), lambda h, t: (h, 0, 0))],
        out_specs=pl.BlockSpec((1, BLK_I, N), lambda h, t: (h, t, 0)),
        out_shape=jax.ShapeDtypeStruct((HEADS, N, N), jnp.float32),
        scratch_shapes=[
            pltpu.VMEM((8, WW), jnp.float32),
            pltpu.VMEM((BLK_I, VW), jnp.float32),
        ],
    )(tab_t)
    return out


# speedup vs baseline: 1.3677x; 1.3677x over previous
import math

import jax
import jax.numpy as jnp
from jax.experimental import pallas as pl
from jax.experimental.pallas import tpu as pltpu

N = 2048
HEADS = 16
NUM_BUCKETS = 32
MAX_DISTANCE = 128
BLK_I = 256
WW = 4352
VW = 4096


def _body(tab_ref, o_ref, w_ref, v128_ref):
    t = pl.program_id(1)

    @pl.when(t == 0)
    def _compute_diag():
        s = jax.lax.broadcasted_iota(jnp.int32, (8, WW), 0)
        z = jax.lax.broadcasted_iota(jnp.int32, (8, WW), 1)
        d = jnp.clip(z - s - (121 + N - 1), -(N - 1), N - 1)
        nb = NUM_BUCKETS // 2
        neg = -d
        ret = jnp.where(neg < 0, nb, 0)
        an = jnp.abs(neg)
        max_exact = nb // 2
        nf = jnp.maximum(an.astype(jnp.float32), 1.0)
        val_large = max_exact + (
            jnp.log(nf / max_exact) / math.log(MAX_DISTANCE / max_exact) * (nb - max_exact)
        ).astype(jnp.int32)
        val_large = jnp.minimum(val_large, nb - 1)
        bucket = ret + jnp.where(an < max_exact, an, val_large)
        acc = jnp.zeros((8, WW), jnp.float32)
        for b in range(NUM_BUCKETS):
            acc = jnp.where(bucket == b, tab_ref[0, 0, b], acc)
        w_ref[:, :] = acc
        for k in range(16):
            v128_ref[8 * k : 8 * k + 8, :] = w_ref[:, 120 - 8 * k : 120 - 8 * k + VW]

    for u in range(2):
        o_ref[0, 128 * u : 128 * u + 128, :] = v128_ref[:, pl.ds((16 - 2 * t - u) * 128, N)]


def kernel(n, relative_attention_bias):
    del n
    tab_t = relative_attention_bias.T.reshape(HEADS, 1, NUM_BUCKETS)
    out = pl.pallas_call(
        _body,
        grid=(HEADS, N // BLK_I),
        in_specs=[pl.BlockSpec((1, 1, NUM_BUCKETS), lambda h, t: (h, 0, 0))],
        out_specs=pl.BlockSpec((1, BLK_I, N), lambda h, t: (h, t, 0)),
        out_shape=jax.ShapeDtypeStruct((HEADS, N, N), jnp.float32),
        scratch_shapes=[
            pltpu.VMEM((8, WW), jnp.float32),
            pltpu.VMEM((128, VW), jnp.float32),
        ],
    )(tab_t)
    return out


# speedup vs baseline: 1.4059x; 1.0279x over previous
import math

import jax
import jax.numpy as jnp
from jax.experimental import pallas as pl
from jax.experimental.pallas import tpu as pltpu

N = 2048
HEADS = 16
NUM_BUCKETS = 32
MAX_DISTANCE = 128
BLK_I = 256
WW = 4352
VW = 4096
_SPANS = ((0, 1152), (1152, 2304), (2304, 3328), (3328, 4352))
_CHUNKS = (range(0, 6), range(6, 11), range(11, 16))


def _bucket_select(tab_ref, w_ref, hh, span):
    z0, z1 = span
    s = jax.lax.broadcasted_iota(jnp.int32, (8, z1 - z0), 0)
    z = jax.lax.broadcasted_iota(jnp.int32, (8, z1 - z0), 1) + z0
    d = jnp.clip(z - s - (121 + N - 1), -(N - 1), N - 1)
    nb = NUM_BUCKETS // 2
    neg = -d
    ret = jnp.where(neg < 0, nb, 0)
    an = jnp.abs(neg)
    max_exact = nb // 2
    nf = jnp.maximum(an.astype(jnp.float32), 1.0)
    val_large = max_exact + (
        jnp.log(nf / max_exact) / math.log(MAX_DISTANCE / max_exact) * (nb - max_exact)
    ).astype(jnp.int32)
    val_large = jnp.minimum(val_large, nb - 1)
    bucket = ret + jnp.where(an < max_exact, an, val_large)
    acc = jnp.zeros((8, z1 - z0), jnp.float32)
    for b in range(NUM_BUCKETS):
        acc = jnp.where(bucket == b, tab_ref[hh, b], acc)
    w_ref[:, z0:z1] = acc


def _expand(w_ref, v2_ref, p, ks):
    for k in ks:
        v2_ref[p, 8 * k : 8 * k + 8, :] = w_ref[:, 120 - 8 * k : 120 - 8 * k + VW]


def _body(tab_ref, o_ref, w_ref, v2_ref):
    h = pl.program_id(0)
    t = pl.program_id(1)
    q = jax.lax.rem(h, 2)

    @pl.when((h == 0) & (t == 0))
    def _build_head0():
        for span in _SPANS:
            _bucket_select(tab_ref, w_ref, 0, span)
        _expand(w_ref, v2_ref, 0, range(16))

    building = h < HEADS - 1
    nq = 1 - q
    for i, span in enumerate(_SPANS):

        @pl.when(building & (t == i + 1))
        def _build_w(span=span):
            _bucket_select(tab_ref, w_ref, h + 1, span)

    for i, ks in enumerate(_CHUNKS):

        @pl.when(building & (t == i + 5))
        def _build_v(ks=ks):
            _expand(w_ref, v2_ref, nq, ks)

    for u in range(2):
        o_ref[0, 128 * u : 128 * u + 128, :] = v2_ref[q, :, pl.ds((16 - 2 * t - u) * 128, N)]


def kernel(n, relative_attention_bias):
    del n
    tab_t = relative_attention_bias.T
    out = pl.pallas_call(
        _body,
        grid=(HEADS, N // BLK_I),
        in_specs=[pl.BlockSpec(memory_space=pltpu.SMEM)],
        out_specs=pl.BlockSpec((1, BLK_I, N), lambda h, t: (h, t, 0)),
        out_shape=jax.ShapeDtypeStruct((HEADS, N, N), jnp.float32),
        scratch_shapes=[
            pltpu.VMEM((8, WW), jnp.float32),
            pltpu.VMEM((2, 128, VW), jnp.float32),
        ],
    )(tab_t)
    return out


# speedup vs baseline: 1.8439x; 1.3115x over previous
import math

import jax
import jax.numpy as jnp
from jax.experimental import pallas as pl
from jax.experimental.pallas import tpu as pltpu

N = 2048
HEADS = 16
NUM_BUCKETS = 32
MAX_DISTANCE = 128
WW = 4352
VW = 4096
NT = N // 128


def _build(tab_ref, w_ref, v3_ref, hh, r):
    s = jax.lax.broadcasted_iota(jnp.int32, (8, WW), 0)
    z = jax.lax.broadcasted_iota(jnp.int32, (8, WW), 1)
    d = jnp.clip(z - s - (121 + N - 1), -(N - 1), N - 1)
    nb = NUM_BUCKETS // 2
    neg = -d
    ret = jnp.where(neg < 0, nb, 0)
    an = jnp.abs(neg)
    max_exact = nb // 2
    nf = jnp.maximum(an.astype(jnp.float32), 1.0)
    val_large = max_exact + (
        jnp.log(nf / max_exact) / math.log(MAX_DISTANCE / max_exact) * (nb - max_exact)
    ).astype(jnp.int32)
    val_large = jnp.minimum(val_large, nb - 1)
    bucket = ret + jnp.where(an < max_exact, an, val_large)
    acc = jnp.zeros((8, WW), jnp.float32)
    for b in range(NUM_BUCKETS):
        acc = jnp.where(bucket == b, tab_ref[hh, b], acc)
    w_ref[:, :] = acc
    for k in range(16):
        v3_ref[r, 8 * k : 8 * k + 8, :] = w_ref[:, 120 - 8 * k : 120 - 8 * k + VW]


def _block_copy(o_ref, v3_ref, sem_ref, h, r, t):
    src = v3_ref.at[r, :, pl.ds((NT - t) * 128, N)]
    dst = o_ref.at[h, pl.ds(128 * t, 128), :]
    return pltpu.make_async_copy(src, dst, sem_ref.at[r])


def _body(tab_ref, o_ref, w_ref, v3_ref, sem_ref):
    h = pl.program_id(0)
    r = jax.lax.rem(h, 3)
    rn = jax.lax.rem(h + 1, 3)

    @pl.when(h == 0)
    def _prologue():
        _build(tab_ref, w_ref, v3_ref, 0, 0)

    for t in range(NT):
        _block_copy(o_ref, v3_ref, sem_ref, h, r, t).start()

    @pl.when(h >= 2)
    def _reclaim():
        for t in range(NT):
            _block_copy(o_ref, v3_ref, sem_ref, h - 2, rn, t).wait()

    @pl.when(h < HEADS - 1)
    def _build_next():
        _build(tab_ref, w_ref, v3_ref, h + 1, rn)

    @pl.when(h == HEADS - 1)
    def _drain():
        for t in range(NT):
            _block_copy(o_ref, v3_ref, sem_ref, h - 1, jax.lax.rem(h - 1, 3), t).wait()
        for t in range(NT):
            _block_copy(o_ref, v3_ref, sem_ref, h, r, t).wait()


def kernel(n, relative_attention_bias):
    del n
    tab_t = relative_attention_bias.T
    out = pl.pallas_call(
        _body,
        grid=(HEADS,),
        in_specs=[pl.BlockSpec(memory_space=pltpu.SMEM)],
        out_specs=pl.BlockSpec(memory_space=pl.ANY),
        out_shape=jax.ShapeDtypeStruct((HEADS, N, N), jnp.float32),
        scratch_shapes=[
            pltpu.VMEM((8, WW), jnp.float32),
            pltpu.VMEM((3, 128, VW), jnp.float32),
            pltpu.SemaphoreType.DMA((3,)),
        ],
    )(tab_t)
    return out
